# baseline (speedup 1.0000x reference)
import functools

import jax
import jax.numpy as jnp
from jax import lax
from jax.experimental import pallas as pl
from jax.experimental.pallas import tpu as pltpu
from jax.experimental.pallas import tpu_sc as plsc

N_X = 1000000
N_Y = 1000000
N_E = 64
B = 16384

NC = 2
NS = 16
L = 16
NW = NC * NS
BPW = B // NW
CHUNK = 128
NCHUNK = BPW // CHUNK
GROUPS = BPW // L

_EPS2 = 1e-24


def _rsqrt(v):
    i = lax.bitcast_convert_type(v, jnp.int32)
    i = jnp.int32(0x5F3759DF) - lax.shift_right_logical(i, 1)
    y = lax.bitcast_convert_type(i, jnp.float32)
    for _ in range(3):
        y = y * (1.5 - 0.5 * v * y * y)
    return y


def _sc_kernel(x_hbm, y_hbm, xt_hbm, yt_hbm, out_hbm,
               xi_v, yi_v, xr_v, yr_v, out_v, sem):
    wid = lax.axis_index("s") * NC + lax.axis_index("c")
    base = wid * BPW

    pltpu.sync_copy(x_hbm.at[pl.ds(base, BPW)], xi_v)
    pltpu.sync_copy(y_hbm.at[pl.ds(base, BPW)], yi_v)

    copies = []
    for c in range(NCHUNK):
        idx = xi_v.at[pl.ds(c * CHUNK, CHUNK)]
        copies.append(pltpu.async_copy(
            xt_hbm.at[idx], xr_v.at[pl.ds(c * CHUNK, CHUNK)], sem))
        idy = yi_v.at[pl.ds(c * CHUNK, CHUNK)]
        copies.append(pltpu.async_copy(
            yt_hbm.at[idy], yr_v.at[pl.ds(c * CHUNK, CHUNK)], sem))
    for cp in copies:
        cp.wait()

    lanes = lax.iota(jnp.int32, L)

    def group_body(g, _):
        rows = g * L + lanes
        zero = jnp.zeros((L,), jnp.float32)
        axy0, axy1 = zero, zero
        axx0, axx1 = zero, zero
        ayy0, ayy1 = zero, zero
        for j in range(N_E):
            col = jnp.full((L,), j, jnp.int32)
            vx = plsc.load_gather(xr_v, [rows, col])
            vy = plsc.load_gather(yr_v, [rows, col])
            if j % 2 == 0:
                axy0 = axy0 + vx * vy
                axx0 = axx0 + vx * vx
                ayy0 = ayy0 + vy * vy
            else:
                axy1 = axy1 + vx * vy
                axx1 = axx1 + vx * vx
                ayy1 = ayy1 + vy * vy
        axy = axy0 + axy1
        axx = axx0 + axx1
        ayy = ayy0 + ayy1
        res = axy * _rsqrt(jnp.maximum(axx, _EPS2)) * _rsqrt(jnp.maximum(ayy, _EPS2))
        plsc.store_scatter(out_v, [rows], res)
        return 0

    lax.fori_loop(0, GROUPS, group_body, 0)

    pltpu.sync_copy(out_v, out_hbm.at[pl.ds(base, BPW)])


@jax.jit
def _run(x, y, x_table, y_table):
    mesh = plsc.VectorSubcoreMesh(core_axis_name="c", subcore_axis_name="s")
    f = functools.partial(
        pl.kernel,
        mesh=mesh,
        out_type=jax.ShapeDtypeStruct((B,), jnp.float32),
        scratch_types=[
            pltpu.VMEM((BPW,), jnp.int32),
            pltpu.VMEM((BPW,), jnp.int32),
            pltpu.VMEM((BPW, N_E), jnp.float32),
            pltpu.VMEM((BPW, N_E), jnp.float32),
            pltpu.VMEM((BPW,), jnp.float32),
            pltpu.SemaphoreType.DMA,
        ],
        compiler_params=pltpu.CompilerParams(
            needs_layout_passes=False, use_tc_tiling_on_sc=False),
    )(_sc_kernel)
    return f(x, y, x_table, y_table)


def kernel(x, y, x_table, y_table):
    return _run(x.astype(jnp.int32), y.astype(jnp.int32), x_table, y_table)


# speedup vs baseline: 1.0028x; 1.0028x over previous
import functools

import jax
import jax.numpy as jnp
from jax import lax
from jax.experimental import pallas as pl
from jax.experimental.pallas import tpu as pltpu
from jax.experimental.pallas import tpu_sc as plsc

N_X = 1000000
N_Y = 1000000
N_E = 64
B = 16384

NC = 2
NS = 16
L = 16
NW = NC * NS
BPW = B // NW
CHUNK = 128
NCHUNK = BPW // CHUNK
GPC = CHUNK // L
FAT = 2 * N_E

_EPS2 = 1e-24


def _rsqrt(v):
    i = lax.bitcast_convert_type(v, jnp.int32)
    i = jnp.int32(0x5F3759DF) - lax.shift_right_logical(i, 1)
    y = lax.bitcast_convert_type(i, jnp.float32)
    for _ in range(3):
        y = y * (1.5 - 0.5 * v * y * y)
    return y


def _sc_kernel(x_hbm, y_hbm, xt_hbm, yt_hbm, out_hbm,
               xi_v, yi_v, xf_v, yf_v, xb0, xb1, yb0, yb1, out_v,
               sem0, sem1):
    wid = lax.axis_index("s") * NC + lax.axis_index("c")
    base = wid * BPW

    pltpu.sync_copy(x_hbm.at[pl.ds(base, BPW)], xi_v)
    pltpu.sync_copy(y_hbm.at[pl.ds(base, BPW)], yi_v)

    for k in range(BPW // L):
        s = pl.ds(k * L, L)
        xf_v[s] = lax.shift_right_logical(xi_v[s], 1)
        yf_v[s] = lax.shift_right_logical(yi_v[s], 1)

    xbufs = (xb0, xb1)
    ybufs = (yb0, yb1)
    sems = (sem0, sem1)

    def fire(c):
        s = pl.ds(c * CHUNK, CHUNK)
        b = c % 2
        return (pltpu.async_copy(xt_hbm.at[xf_v.at[s]], xbufs[b], sems[b]),
                pltpu.async_copy(yt_hbm.at[yf_v.at[s]], ybufs[b], sems[b]))

    lanes = lax.iota(jnp.int32, L)
    inflight = fire(0)
    for c in range(NCHUNK):
        for cp in inflight:
            cp.wait()
        if c + 1 < NCHUNK:
            nxt = fire(c + 1)
        xb = xbufs[c % 2]
        yb = ybufs[c % 2]
        s = pl.ds(c * CHUNK, CHUNK)

        def group_body(g, _, xb=xb, yb=yb, c=c):
            rows = g * L + lanes
            xhalf = plsc.load_gather(xi_v, [c * CHUNK + rows])
            yhalf = plsc.load_gather(yi_v, [c * CHUNK + rows])
            xoff = lax.shift_left(jnp.bitwise_and(xhalf, 1), 6)
            yoff = lax.shift_left(jnp.bitwise_and(yhalf, 1), 6)
            zero = jnp.zeros((L,), jnp.float32)
            axy0, axy1 = zero, zero
            axx0, axx1 = zero, zero
            ayy0, ayy1 = zero, zero
            for j in range(N_E):
                vx = plsc.load_gather(xb, [rows, xoff + j])
                vy = plsc.load_gather(yb, [rows, yoff + j])
                if j % 2 == 0:
                    axy0 = axy0 + vx * vy
                    axx0 = axx0 + vx * vx
                    ayy0 = ayy0 + vy * vy
                else:
                    axy1 = axy1 + vx * vy
                    axx1 = axx1 + vx * vx
                    ayy1 = ayy1 + vy * vy
            axy = axy0 + axy1
            axx = axx0 + axx1
            ayy = ayy0 + ayy1
            res = (axy * _rsqrt(jnp.maximum(axx, _EPS2))
                       * _rsqrt(jnp.maximum(ayy, _EPS2)))
            plsc.store_scatter(out_v, [c * CHUNK + rows], res)
            return 0

        lax.fori_loop(0, GPC, group_body, 0)
        if c + 1 < NCHUNK:
            inflight = nxt

    pltpu.sync_copy(out_v, out_hbm.at[pl.ds(base, BPW)])


@jax.jit
def _run(x, y, x_table, y_table):
    mesh = plsc.VectorSubcoreMesh(core_axis_name="c", subcore_axis_name="s")
    f = functools.partial(
        pl.kernel,
        mesh=mesh,
        out_type=jax.ShapeDtypeStruct((B,), jnp.float32),
        scratch_types=[
            pltpu.VMEM((BPW,), jnp.int32),
            pltpu.VMEM((BPW,), jnp.int32),
            pltpu.VMEM((BPW,), jnp.int32),
            pltpu.VMEM((BPW,), jnp.int32),
            pltpu.VMEM((CHUNK, FAT), jnp.float32),
            pltpu.VMEM((CHUNK, FAT), jnp.float32),
            pltpu.VMEM((CHUNK, FAT), jnp.float32),
            pltpu.VMEM((CHUNK, FAT), jnp.float32),
            pltpu.VMEM((BPW,), jnp.float32),
            pltpu.SemaphoreType.DMA,
            pltpu.SemaphoreType.DMA,
        ],
        compiler_params=pltpu.CompilerParams(
            needs_layout_passes=False, use_tc_tiling_on_sc=True),
    )(_sc_kernel)
    xt_fat = x_table.reshape(N_X // 2, FAT)
    yt_fat = y_table.reshape(N_Y // 2, FAT)
    return f(x, y, xt_fat, yt_fat)


def kernel(x, y, x_table, y_table):
    return _run(x.astype(jnp.int32), y.astype(jnp.int32), x_table, y_table)
